# baseline (speedup 1.0000x reference)
import functools
import math

import jax
import jax.numpy as jnp
from jax import lax
from jax.experimental import pallas as pl
from jax.experimental.pallas import tpu as pltpu
from jax.experimental.pallas import tpu_sc as plsc

_B = 32
_N = 32768
_D = 64
_K = 16
_L = 16
_NC = 2
_NS = 16
_CHUNKS = _N // _L
_GROUP = 8
_HID = 1024
_W = 128
_GB = 8
_NG = _B // _GB


_NACC = 8
_GC = 32
_NGRP = _CHUNKS // _GC
_CAP = 256


def _sc_topk_body(act_hbm, vals_hbm, idx_hbm,
                  acts_v, gmax_v, bufv_v, bufi_v, out_v, outi_v):
    wid = lax.axis_index("s") * _NC + lax.axis_index("c")
    pltpu.sync_copy(act_hbm.at[wid], acts_v)

    ids16 = lax.iota(jnp.int32, _L)

    def load_chunk(off):
        return acts_v[pl.ds(off, _L)]

    def sort_desc(x):
        return plsc.sort_key_val(x, x, descending=True)[0]

    def step8(accs, base):
        vs = [load_chunk(base + j * _L) for j in range(_NACC)]
        accs = [jnp.sort(jnp.maximum(accs[j], sort_desc(vs[j])))
                for j in range(_NACC)]
        m = vs[0]
        for j in range(1, _NACC):
            m = jnp.maximum(m, vs[j])
        return accs, m

    init = [load_chunk(j * _L) for j in range(_NACC)]
    accs = [jnp.sort(v) for v in init]
    gm = init[0]
    for j in range(1, _NACC):
        gm = jnp.maximum(gm, init[j])
    for s in range(1, _GC // _NACC):
        accs, m = step8(accs, s * _NACC * _L)
        gm = jnp.maximum(gm, m)
    gmax_v[pl.ds(0, _L)] = gm

    @plsc.parallel_loop(1, _NGRP, step=1, carry=tuple(accs))
    def _p1(g, accs):
        accs = list(accs)
        base = g * _GC * _L
        gm = None
        for s in range(_GC // _NACC):
            accs, m = step8(accs, base + s * _NACC * _L)
            gm = m if gm is None else jnp.maximum(gm, m)
        gmax_v[pl.ds(g * _L, _L)] = gm
        return tuple(accs)

    accs = list(_p1)

    while len(accs) > 1:
        accs = [jnp.sort(jnp.maximum(accs[a], lax.rev(accs[a + 1], (0,))))
                for a in range(0, len(accs), 2)]
    T = jnp.min(accs[0])

    for i in range(_CAP // _L):
        bufv_v[pl.ds(i * _L, _L)] = jnp.full((_L,), -1.0, jnp.float32)

    def append_chunk(base_cnt, off):
        v = load_chunk(off)
        mask = v >= T

        def app(bc):
            cs = jnp.cumsum(mask.astype(jnp.int32))
            pos = jnp.minimum(bc + cs - 1, _CAP - 1)
            plsc.store_scatter(bufv_v, [pos], v, mask=mask)
            plsc.store_scatter(bufi_v, [pos], ids16 + off, mask=mask)
            return bc + jnp.max(cs)

        return lax.cond(jnp.any(mask), app, lambda b: b, base_cnt)

    def group2(g, base_cnt):
        gm = gmax_v[pl.ds(g * _L, _L)]

        def refine(bc):
            gbase = g * _GC * _L
            for s in range(_GC // _NACC):
                vs = [load_chunk(gbase + (s * _NACC + j) * _L)
                      for j in range(_NACC)]
                sm = vs[0]
                for j in range(1, _NACC):
                    sm = jnp.maximum(sm, vs[j])

                def ref2(bc2, s=s, gbase=gbase):
                    for j in range(_NACC):
                        bc2 = append_chunk(bc2, gbase + (s * _NACC + j) * _L)
                    return bc2

                bc = lax.cond(jnp.any(sm >= T), ref2, lambda b: b, bc)
            return bc

        return lax.cond(jnp.any(gm >= T), refine, lambda b: b, base_cnt)

    lax.fori_loop(0, _NGRP, group2, jnp.int32(0))

    m_gt = jnp.int32(0)
    for i in range(_CAP // _L):
        bv = bufv_v[pl.ds(i * _L, _L)]
        m_gt = m_gt + jnp.sum((bv > T).astype(jnp.int32))
    need_eq = 16 - m_gt

    obase = jnp.int32(0)
    eqbase = jnp.int32(0)
    for i in range(_CAP // _L):
        bv = bufv_v[pl.ds(i * _L, _L)]
        bi = bufi_v[pl.ds(i * _L, _L)]
        gt = bv > T
        eq = bv == T
        eqcs = jnp.cumsum(eq.astype(jnp.int32))
        keep = gt | (eq & ((eqbase + eqcs) <= need_eq))
        cnt = jnp.sum(keep.astype(jnp.int32))
        plsc.store_compressed(out_v.at[pl.ds(obase, _L)], bv, mask=keep)
        plsc.store_compressed(outi_v.at[pl.ds(obase, _L)], bi, mask=keep)
        obase = obase + cnt
        eqbase = eqbase + jnp.max(eqcs)

    pltpu.sync_copy(out_v.at[pl.ds(0, _K)], vals_hbm.at[wid])
    pltpu.sync_copy(outi_v.at[pl.ds(0, _K)], idx_hbm.at[wid])


def _sc_topk(activations):
    mesh = plsc.VectorSubcoreMesh(core_axis_name="c", subcore_axis_name="s")
    fn = pl.kernel(
        _sc_topk_body,
        mesh=mesh,
        compiler_params=pltpu.CompilerParams(
            needs_layout_passes=False, use_tc_tiling_on_sc=False),
        out_type=[
            jax.ShapeDtypeStruct((_B, _K), jnp.float32),
            jax.ShapeDtypeStruct((_B, _K), jnp.int32),
        ],
        scratch_types=[
            pltpu.VMEM((_N,), jnp.float32),
            pltpu.VMEM((_NGRP * _L,), jnp.float32),
            pltpu.VMEM((_CAP,), jnp.float32),
            pltpu.VMEM((_CAP,), jnp.int32),
            pltpu.VMEM((_CAP + _L,), jnp.float32),
            pltpu.VMEM((_CAP + _L,), jnp.int32),
        ],
    )
    return fn(activations)


_NSLOT = 12


def _sc_gather_body(idx_hbm, tok_hbm, out_hbm, idx_v, blk_v, tok_v, sem):
    b = lax.axis_index("s") * _NC + lax.axis_index("c")
    pltpu.sync_copy(idx_hbm.at[b], idx_v)
    d16 = lax.iota(jnp.int32, _L)
    vi = idx_v[...]
    nks = [jnp.sum(jnp.where(d16 == k, vi, 0)) for k in range(_K)]

    def copy_k(k, slot):
        blk = nks[k] // _W
        return pltpu.make_async_copy(
            tok_hbm.at[b, :, pl.ds(blk * _W, _W)],
            blk_v.at[slot], sem.at[slot])

    for k in range(_NSLOT):
        copy_k(k, k).start()
    for k in range(_K):
        slot = k % _NSLOT
        copy_k(k, slot).wait()
        mod = nks[k] % _W
        for g in range(_D // _L):
            vals = plsc.load_gather(
                blk_v.at[slot], [d16 + g * _L, d16 * 0 + mod])
            tok_v[k, pl.ds(g * _L, _L)] = vals
        if k + _NSLOT < _K:
            copy_k(k + _NSLOT, slot).start()
    pltpu.sync_copy(tok_v, out_hbm.at[b])


def _sc_gather(idx, tok_t):
    mesh = plsc.VectorSubcoreMesh(core_axis_name="c", subcore_axis_name="s")
    fn = pl.kernel(
        _sc_gather_body,
        mesh=mesh,
        compiler_params=pltpu.CompilerParams(
            needs_layout_passes=False, use_tc_tiling_on_sc=True),
        out_type=[
            jax.ShapeDtypeStruct((_B, _K, _D), jnp.float32),
        ],
        scratch_types=[
            pltpu.VMEM((_K,), jnp.int32),
            pltpu.VMEM((_NSLOT, _D, _W), jnp.float32),
            pltpu.VMEM((_K, _D), jnp.float32),
            pltpu.SemaphoreType.DMA((_NSLOT,)),
        ],
    )
    return fn(idx, tok_t)[0]


def _stats_mlp_body(vals_ref, tok_ref,
                    w1_ref, b1_ref, w2_ref, b2_ref, out_ref):
    t = tok_ref[...]
    act = vals_ref[...]

    mass = jnp.sum(act, axis=1)
    dn = jnp.maximum(mass, 1.0)
    w = t * act[:, :, None]
    centroid = jnp.sum(w, axis=1) / dn[:, None]
    diffs = t - centroid[:, None, :]

    d4 = t[:, :, None, :] - t[:, None, :, :]
    d2 = jnp.sum(d4 * d4, axis=-1)
    d2 = jnp.maximum(d2, 0.0)
    pairwise = jnp.where(d2 > 0, jnp.sqrt(jnp.where(d2 > 0, d2, 1.0)), 0.0)

    row_i = lax.broadcasted_iota(jnp.int32, (_K, _K), 0)
    col_i = lax.broadcasted_iota(jnp.int32, (_K, _K), 1)
    tri = (col_i > row_i).astype(jnp.float32)[None]

    pw = act[:, :, None] * act[:, None, :] * tri
    wp = pairwise * pw
    pm = jnp.maximum(jnp.sum(jnp.sum(pw, axis=2), axis=1), 1.0)
    mean_pair = jnp.sum(jnp.sum(wp, axis=2), axis=1) / pm
    max_pair = jnp.max(jnp.max(wp, axis=2), axis=1)
    pc = (pairwise - mean_pair[:, None, None]) * pw
    pair_var = jnp.maximum(jnp.sum(jnp.sum(pc * pc, axis=2), axis=1) / pm, 0.0)
    pair_std = jnp.sqrt(pair_var + 1e-06)

    disp = jnp.sqrt(jnp.sum(diffs * diffs, axis=-1) + 1e-06)
    wd = disp * act
    mean_disp = jnp.sum(wd, axis=1) / dn
    max_disp = jnp.max(wd, axis=1)
    dc = (disp - mean_disp[:, None]) * act
    disp_var = jnp.maximum(jnp.sum(dc * dc, axis=1) / dn, 0.0)
    disp_std = jnp.sqrt(disp_var + 1e-06)

    support_ratio = jnp.mean((act > 0.001).astype(jnp.float32), axis=1)
    activation_mean = jnp.mean(act, axis=1)
    act_dev = act - activation_mean[:, None]
    activation_std = jnp.sqrt(jnp.mean(act_dev * act_dev, axis=1))
    centroid_norm = jnp.sqrt(jnp.sum(centroid * centroid, axis=1) + 1e-06)
    token_norm = jnp.sqrt(jnp.sum(t * t, axis=-1) + 1e-06)
    token_norm_mean = jnp.sum(token_norm * act, axis=1) / dn
    second_moment = jnp.sqrt(
        jnp.sum(jnp.sum(w * w, axis=2), axis=1) / dn + 1e-06)

    summary = jnp.stack(
        [mean_pair, max_pair, pair_std, mean_disp, max_disp, disp_std,
         support_ratio, activation_mean, activation_std, centroid_norm,
         token_norm_mean, second_moment], axis=-1)

    h = lax.dot_general(summary, w1_ref[...],
                        (((1,), (1,)), ((), ())),
                        preferred_element_type=jnp.float32) + b1_ref[...]
    h = 0.5 * h * (1.0 + lax.erf(h * (1.0 / math.sqrt(2.0))))
    out_ref[...] = lax.dot_general(h, w2_ref[...],
                                   (((1,), (1,)), ((), ())),
                                   preferred_element_type=jnp.float32) \
        + b2_ref[...]


def _stats_mlp(vals, toks, W1, b1, W2, b2, interpret=False):
    return pl.pallas_call(
        _stats_mlp_body,
        out_shape=jax.ShapeDtypeStruct((_B, _HID), jnp.float32),
        interpret=interpret,
    )(vals, toks, W1, b1, W2, b2)


def kernel(lifted_tokens, activations, W1, b1, W2, b2):
    vals, idx = _sc_topk(activations)
    tok_t = jnp.transpose(lifted_tokens, (0, 2, 1))
    toks = _sc_gather(idx, tok_t)
    return _stats_mlp(vals, toks, W1, b1, W2, b2)


# speedup vs baseline: 1.0046x; 1.0046x over previous
import functools
import math

import jax
import jax.numpy as jnp
from jax import lax
from jax.experimental import pallas as pl
from jax.experimental.pallas import tpu as pltpu
from jax.experimental.pallas import tpu_sc as plsc

_B = 32
_N = 32768
_D = 64
_K = 16
_L = 16
_NC = 2
_NS = 16
_CHUNKS = _N // _L
_GROUP = 8
_HID = 1024
_W = 128
_GB = 8
_NG = _B // _GB


_NACC = 8
_GC = 32
_NGRP = _CHUNKS // _GC
_CAP = 256


def _sc_topk_body(act_hbm, vals_hbm, idx_hbm,
                  acts_v, gmax_v, bufv_v, bufi_v, out_v, outi_v):
    wid = lax.axis_index("s") * _NC + lax.axis_index("c")
    pltpu.sync_copy(act_hbm.at[wid], acts_v)

    ids16 = lax.iota(jnp.int32, _L)

    def load_chunk(off):
        return acts_v[pl.ds(off, _L)]

    def sort_desc(x):
        return plsc.sort_key_val(x, x, descending=True)[0]

    def step8(accs, base):
        vs = [load_chunk(base + j * _L) for j in range(_NACC)]
        accs = [jnp.sort(jnp.maximum(accs[j], sort_desc(vs[j])))
                for j in range(_NACC)]
        m = vs[0]
        for j in range(1, _NACC):
            m = jnp.maximum(m, vs[j])
        return accs, m

    init = [load_chunk(j * _L) for j in range(_NACC)]
    accs = [jnp.sort(v) for v in init]
    gm = init[0]
    for j in range(1, _NACC):
        gm = jnp.maximum(gm, init[j])
    for s in range(1, _GC // _NACC):
        accs, m = step8(accs, s * _NACC * _L)
        gm = jnp.maximum(gm, m)
    gmax_v[pl.ds(0, _L)] = gm

    @plsc.parallel_loop(1, _NGRP, step=1, carry=tuple(accs))
    def _p1(g, accs):
        accs = list(accs)
        base = g * _GC * _L
        gm = None
        for s in range(_GC // _NACC):
            accs, m = step8(accs, base + s * _NACC * _L)
            gm = m if gm is None else jnp.maximum(gm, m)
        gmax_v[pl.ds(g * _L, _L)] = gm
        return tuple(accs)

    accs = list(_p1)

    while len(accs) > 1:
        accs = [jnp.sort(jnp.maximum(accs[a], lax.rev(accs[a + 1], (0,))))
                for a in range(0, len(accs), 2)]
    T = jnp.min(accs[0])

    for i in range(_CAP // _L):
        bufv_v[pl.ds(i * _L, _L)] = jnp.full((_L,), -1.0, jnp.float32)

    def append_chunk(base_cnt, off):
        v = load_chunk(off)
        mask = v >= T

        def app(bc):
            cs = jnp.cumsum(mask.astype(jnp.int32))
            pos = jnp.minimum(bc + cs - 1, _CAP - 1)
            plsc.store_scatter(bufv_v, [pos], v, mask=mask)
            plsc.store_scatter(bufi_v, [pos], ids16 + off, mask=mask)
            return bc + jnp.max(cs)

        return lax.cond(jnp.any(mask), app, lambda b: b, base_cnt)

    def group2(g, base_cnt):
        gm = gmax_v[pl.ds(g * _L, _L)]

        def refine(bc):
            gbase = g * _GC * _L
            for s in range(_GC // _NACC):
                vs = [load_chunk(gbase + (s * _NACC + j) * _L)
                      for j in range(_NACC)]
                sm = vs[0]
                for j in range(1, _NACC):
                    sm = jnp.maximum(sm, vs[j])

                def ref2(bc2, s=s, gbase=gbase):
                    for j in range(_NACC):
                        bc2 = append_chunk(bc2, gbase + (s * _NACC + j) * _L)
                    return bc2

                bc = lax.cond(jnp.any(sm >= T), ref2, lambda b: b, bc)
            return bc

        return lax.cond(jnp.any(gm >= T), refine, lambda b: b, base_cnt)

    lax.fori_loop(0, _NGRP, group2, jnp.int32(0))

    m_gt = jnp.int32(0)
    for i in range(_CAP // _L):
        bv = bufv_v[pl.ds(i * _L, _L)]
        m_gt = m_gt + jnp.sum((bv > T).astype(jnp.int32))
    need_eq = 16 - m_gt

    obase = jnp.int32(0)
    eqbase = jnp.int32(0)
    for i in range(_CAP // _L):
        bv = bufv_v[pl.ds(i * _L, _L)]
        bi = bufi_v[pl.ds(i * _L, _L)]
        gt = bv > T
        eq = bv == T
        eqcs = jnp.cumsum(eq.astype(jnp.int32))
        keep = gt | (eq & ((eqbase + eqcs) <= need_eq))
        cnt = jnp.sum(keep.astype(jnp.int32))
        plsc.store_compressed(out_v.at[pl.ds(obase, _L)], bv, mask=keep)
        plsc.store_compressed(outi_v.at[pl.ds(obase, _L)], bi, mask=keep)
        obase = obase + cnt
        eqbase = eqbase + jnp.max(eqcs)

    pltpu.sync_copy(out_v.at[pl.ds(0, _K)], vals_hbm.at[wid])
    pltpu.sync_copy(outi_v.at[pl.ds(0, _K)], idx_hbm.at[wid])


def _sc_topk(activations):
    mesh = plsc.VectorSubcoreMesh(core_axis_name="c", subcore_axis_name="s")
    fn = pl.kernel(
        _sc_topk_body,
        mesh=mesh,
        compiler_params=pltpu.CompilerParams(
            needs_layout_passes=False, use_tc_tiling_on_sc=False),
        out_type=[
            jax.ShapeDtypeStruct((_B, _K), jnp.float32),
            jax.ShapeDtypeStruct((_B, _K), jnp.int32),
        ],
        scratch_types=[
            pltpu.VMEM((_N,), jnp.float32),
            pltpu.VMEM((_NGRP * _L,), jnp.float32),
            pltpu.VMEM((_CAP,), jnp.float32),
            pltpu.VMEM((_CAP,), jnp.int32),
            pltpu.VMEM((_CAP + _L,), jnp.float32),
            pltpu.VMEM((_CAP + _L,), jnp.int32),
        ],
    )
    return fn(activations)


_NSLOT = 8


def _sc_gather_body(idx_hbm, tok_hbm, out_hbm, idx_v, blk_v, tok_v, sem):
    b = lax.axis_index("s") * _NC + lax.axis_index("c")
    pltpu.sync_copy(idx_hbm.at[b], idx_v)
    d16 = lax.iota(jnp.int32, _L)
    vi = idx_v[...]
    nks = [jnp.sum(jnp.where(d16 == k, vi, 0)) for k in range(_K)]

    def copy_k(k, slot):
        blk = nks[k] // _W
        return pltpu.make_async_copy(
            tok_hbm.at[b, :, pl.ds(blk * _W, _W)],
            blk_v.at[slot], sem.at[slot])

    for k in range(_NSLOT):
        copy_k(k, k).start()
    for k in range(_K):
        slot = k % _NSLOT
        copy_k(k, slot).wait()
        mod = nks[k] % _W
        for g in range(_D // _L):
            vals = plsc.load_gather(
                blk_v.at[slot], [d16 + g * _L, d16 * 0 + mod])
            tok_v[k, pl.ds(g * _L, _L)] = vals
        if k + _NSLOT < _K:
            copy_k(k + _NSLOT, slot).start()
    pltpu.sync_copy(tok_v, out_hbm.at[b])


def _sc_gather(idx, tok_t):
    mesh = plsc.VectorSubcoreMesh(core_axis_name="c", subcore_axis_name="s")
    fn = pl.kernel(
        _sc_gather_body,
        mesh=mesh,
        compiler_params=pltpu.CompilerParams(
            needs_layout_passes=False, use_tc_tiling_on_sc=True),
        out_type=[
            jax.ShapeDtypeStruct((_B, _K, _D), jnp.float32),
        ],
        scratch_types=[
            pltpu.VMEM((_K,), jnp.int32),
            pltpu.VMEM((_NSLOT, _D, _W), jnp.float32),
            pltpu.VMEM((_K, _D), jnp.float32),
            pltpu.SemaphoreType.DMA((_NSLOT,)),
        ],
    )
    return fn(idx, tok_t)[0]


def _stats_mlp_body(vals_ref, tok_ref,
                    w1_ref, b1_ref, w2_ref, b2_ref, out_ref):
    t = tok_ref[...]
    act = vals_ref[...]

    mass = jnp.sum(act, axis=1)
    dn = jnp.maximum(mass, 1.0)
    w = t * act[:, :, None]
    centroid = jnp.sum(w, axis=1) / dn[:, None]
    diffs = t - centroid[:, None, :]

    d4 = t[:, :, None, :] - t[:, None, :, :]
    d2 = jnp.sum(d4 * d4, axis=-1)
    d2 = jnp.maximum(d2, 0.0)
    pairwise = jnp.where(d2 > 0, jnp.sqrt(jnp.where(d2 > 0, d2, 1.0)), 0.0)

    row_i = lax.broadcasted_iota(jnp.int32, (_K, _K), 0)
    col_i = lax.broadcasted_iota(jnp.int32, (_K, _K), 1)
    tri = (col_i > row_i).astype(jnp.float32)[None]

    pw = act[:, :, None] * act[:, None, :] * tri
    wp = pairwise * pw
    pm = jnp.maximum(jnp.sum(jnp.sum(pw, axis=2), axis=1), 1.0)
    mean_pair = jnp.sum(jnp.sum(wp, axis=2), axis=1) / pm
    max_pair = jnp.max(jnp.max(wp, axis=2), axis=1)
    pc = (pairwise - mean_pair[:, None, None]) * pw
    pair_var = jnp.maximum(jnp.sum(jnp.sum(pc * pc, axis=2), axis=1) / pm, 0.0)
    pair_std = jnp.sqrt(pair_var + 1e-06)

    disp = jnp.sqrt(jnp.sum(diffs * diffs, axis=-1) + 1e-06)
    wd = disp * act
    mean_disp = jnp.sum(wd, axis=1) / dn
    max_disp = jnp.max(wd, axis=1)
    dc = (disp - mean_disp[:, None]) * act
    disp_var = jnp.maximum(jnp.sum(dc * dc, axis=1) / dn, 0.0)
    disp_std = jnp.sqrt(disp_var + 1e-06)

    support_ratio = jnp.mean((act > 0.001).astype(jnp.float32), axis=1)
    activation_mean = jnp.mean(act, axis=1)
    act_dev = act - activation_mean[:, None]
    activation_std = jnp.sqrt(jnp.mean(act_dev * act_dev, axis=1))
    centroid_norm = jnp.sqrt(jnp.sum(centroid * centroid, axis=1) + 1e-06)
    token_norm = jnp.sqrt(jnp.sum(t * t, axis=-1) + 1e-06)
    token_norm_mean = jnp.sum(token_norm * act, axis=1) / dn
    second_moment = jnp.sqrt(
        jnp.sum(jnp.sum(w * w, axis=2), axis=1) / dn + 1e-06)

    summary = jnp.stack(
        [mean_pair, max_pair, pair_std, mean_disp, max_disp, disp_std,
         support_ratio, activation_mean, activation_std, centroid_norm,
         token_norm_mean, second_moment], axis=-1)

    h = lax.dot_general(summary, w1_ref[...],
                        (((1,), (1,)), ((), ())),
                        preferred_element_type=jnp.float32) + b1_ref[...]
    h = 0.5 * h * (1.0 + lax.erf(h * (1.0 / math.sqrt(2.0))))
    out_ref[...] = lax.dot_general(h, w2_ref[...],
                                   (((1,), (1,)), ((), ())),
                                   preferred_element_type=jnp.float32) \
        + b2_ref[...]


def _stats_mlp(vals, toks, W1, b1, W2, b2, interpret=False):
    return pl.pallas_call(
        _stats_mlp_body,
        out_shape=jax.ShapeDtypeStruct((_B, _HID), jnp.float32),
        interpret=interpret,
    )(vals, toks, W1, b1, W2, b2)


def kernel(lifted_tokens, activations, W1, b1, W2, b2):
    vals, idx = _sc_topk(activations)
    tok_t = jnp.transpose(lifted_tokens, (0, 2, 1))
    toks = _sc_gather(idx, tok_t)
    return _stats_mlp(vals, toks, W1, b1, W2, b2)
